# Initial kernel scaffold; baseline (speedup 1.0000x reference)
import jax
import jax.numpy as jnp
from jax.experimental import pallas as pl


def kernel(word_id, topic_id, src_ww, dst_ww, w_ww, src_wt, dst_wt, w_wt, src_wd, dst_wd, w_wd, src_td, dst_td, w_td, src_tt, dst_tt, w_tt, word_embeds, topic_embeds, l1_Www, l1_bww, l1_Wwt, l1_bwt, l1_Wwd, l1_bwd, l1_Wtd, l1_btd, l1_Wtt, l1_btt, l2_Www, l2_bww, l2_Wwt, l2_bwt, l2_Wwd, l2_bwd, l2_Wtd, l2_btd, l2_Wtt, l2_btt, out_W, out_b, y_data):
    raise NotImplementedError("write your pallas kernel here")


# speedup vs baseline: 18.2798x; 18.2798x over previous
import functools

import jax
import jax.numpy as jnp
from jax import lax
from jax.experimental import pallas as pl
from jax.experimental.pallas import tpu as pltpu
from jax.experimental.pallas import tpu_sc as plsc

Nw, Nt, Nd = 40000, 800, 3200
B = 16
NWORD, NTOPIC = 15000, 56

R_WW, R_WT, R_WD, R_TD, R_TT = 2048, 1024, 1024, 256, 128
NSUB = 16

AW, AT, AD = 40064, 816, 3264
DUM_W, DUM_T, DUM_D = 40000, 800, 3200


def _pre_body(we, te, W1ww, b1ww, W1wt, b1wt, W1wd, b1wd, W1td, b1td,
              W1tt, b1tt, W2ww, b2ww, W2wt, b2wt, W2wd, b2wd, W2td, b2td,
              W2tt, b2tt, outW, wpa_o, wpb_o, tpp_o, cv_o):
    dot = lambda a, b: jnp.dot(a, b, preferred_element_type=jnp.float32)
    u = outW[...]
    p2 = dot(W2wd[...], u)
    q2 = dot(W2wt[...], p2)
    cA = dot(b2wt[...], p2) + dot(b2wd[...], u)
    r2 = dot(W2ww[...], q2)
    cB = dot(b2ww[...], q2)
    p1 = dot(W1wd[...], r2)
    q1 = dot(W1wt[...], p1)
    cC = dot(b1wt[...], p1) + dot(b1wd[...], r2)
    r1 = dot(W1ww[...], q1)
    cD = dot(b1ww[...], q1)
    pt = dot(W2tt[...], u)
    qt = dot(W2td[...], pt)
    cE = dot(b2td[...], pt) + dot(b2tt[...], u)
    pb = dot(W1wd[...], qt)
    qb = dot(W1wt[...], pb)
    cF = dot(b1wt[...], pb) + dot(b1wd[...], qt)
    rbv = dot(W1ww[...], qb)
    cG = dot(b1ww[...], qb)
    pc = dot(W1tt[...], qt)
    qc = dot(W1td[...], pc)
    cH = dot(b1td[...], pc) + dot(b1tt[...], qt)

    wpa_o[...] = dot(we[...], r1) + cD
    wpb_o[...] = dot(we[...], rbv) + cG
    tpp_o[...] = dot(te[...], qc) + cH
    lane = lax.broadcasted_iota(jnp.int32, (1, 128), 1)
    c2 = cB + cC
    cv = (jnp.where(lane == 0, c2, 0.0) + jnp.where(lane == 1, cF, 0.0)
          + jnp.where(lane == 2, cE, 0.0) + jnp.where(lane == 3, cA, 0.0))
    cv_o[...] = cv


def _post_body(sd, y, ob, loss_o, probs_o):
    pooled = jnp.mean(sd[...], axis=1, keepdims=True)
    lg = pooled + ob[0, 0]
    t = y[...]
    terms = jnp.maximum(lg, 0.0) - lg * t + jnp.log1p(jnp.exp(-jnp.abs(lg)))
    loss_o[...] = jnp.mean(terms, keepdims=True)
    probs_o[...] = jax.nn.sigmoid(lg)


def _sc_body(wpa, wpb, tpp, cvec, wid_h, tid_h,
             wws, wwd, www, wts, wtd, wtw, wds, wdd, wdw,
             tds, tdd, tdw, tts, ttd, ttw,
             out_h,
             srcA_v, srcB_v, srcT_v, cv_v, zbuf, idb, fb0, fb1, fb2,
             sidx2, didx2, wv2, msgA, msgB, ones_r,
             S_SRCA, S_SRCB, S_SRC2, S_SRCWT, S_SRCW,
             S_ACCA, S_ACCB, S_CNTWW, S_ACC2,
             S_SRCTT, S_ACCWT, S_CNTWT, S_ACCTT, S_CNTTT, S_SRCT,
             S_ACCD, S_CNTWD, S_ACCTD, S_CNTTD):
    cid = lax.axis_index("c")
    wid = lax.axis_index("s")
    act = cid == 0
    ds = pl.ds

    def agg_pass(s2d, d2d, w2d, rows_pw, nch, srcrefs, accrefs, cntref):
        rowbase = wid * rows_pw
        msgs = (msgA, msgB)

        def chunk(ch, _):
            rb = rowbase + ch * 8
            pltpu.sync_copy(s2d.at[ds(rb, 8)], sidx2)
            pltpu.sync_copy(d2d.at[ds(rb, 8)], didx2)
            pltpu.sync_copy(w2d.at[ds(rb, 8)], wv2)
            for j in range(8):
                srow = sidx2.at[j]
                wrow = wv2.at[j]
                for k, srcv in enumerate(srcrefs):
                    mrow = msgs[k].at[j]
                    for i in range(8):
                        sl = ds(i * 16, 16)
                        idx = srow[sl]
                        mrow[sl] = plsc.load_gather(srcv, [idx]) * wrow[sl]
            for j in range(8):
                drow = didx2.at[j]
                for k, acc in enumerate(accrefs):
                    pltpu.sync_copy(msgs[k].at[j], acc.at[drow], add=True)
                if cntref is not None:
                    pltpu.sync_copy(ones_r, cntref.at[drow], add=True)
            return 0

        lax.fori_loop(0, nch, chunk, 0)

    @pl.when(act)
    def _p0():
        def zb(i, _):
            zbuf[ds(i * 16, 16)] = jnp.zeros((16,), jnp.float32)
            return 0
        lax.fori_loop(0, AD // 16, zb, 0)
        for i in range(8):
            ones_r[ds(i * 16, 16)] = jnp.ones((16,), jnp.float32)
        pltpu.sync_copy(cvec, cv_v)

        zb2 = wid * 2504
        for tbl in (S_ACCA, S_ACCB, S_CNTWW, S_ACC2):
            pltpu.sync_copy(zbuf.at[ds(0, 2504)], tbl.at[ds(zb2, 2504)])
        for w_i, tbl in ((0, S_ACCD), (1, S_CNTWD), (2, S_ACCTD),
                         (3, S_CNTTD)):
            @pl.when(wid == w_i)
            def _z():
                pltpu.sync_copy(zbuf, tbl)
        for w_i, tbl in ((4, S_ACCWT), (5, S_CNTWT), (6, S_ACCTT),
                         (7, S_CNTTT)):
            @pl.when(wid == w_i)
            def _z():
                pltpu.sync_copy(zbuf.at[ds(0, AT)], tbl)

        @pl.when(wid < 10)
        def _gw():
            pltpu.sync_copy(wpa, srcA_v.at[ds(0, NWORD)])
            pltpu.sync_copy(wpb, srcB_v.at[ds(0, NWORD)])
            base = wid * 4000
            pltpu.sync_copy(wid_h.at[ds(base, 4000)], idb)

            def g(i, _):
                sl = ds(i * 16, 16)
                idx = idb[sl]
                fb0[sl] = plsc.load_gather(srcA_v, [idx])
                fb1[sl] = plsc.load_gather(srcB_v, [idx])
                return 0
            lax.fori_loop(0, 250, g, 0)
            pltpu.sync_copy(fb0, S_SRCA.at[ds(base, 4000)])
            pltpu.sync_copy(fb1, S_SRCB.at[ds(base, 4000)])

        @pl.when(wid == 10)
        def _gt():
            pltpu.sync_copy(tpp, srcT_v.at[ds(0, NTOPIC)])
            pltpu.sync_copy(tid_h, idb.at[ds(0, 800)])

            def g(i, _):
                sl = ds(i * 16, 16)
                fb0[sl] = plsc.load_gather(srcT_v, [idb[sl]])
                return 0
            lax.fori_loop(0, 50, g, 0)
            pltpu.sync_copy(fb0.at[ds(0, 800)], S_SRCTT)

    plsc.subcore_barrier()

    @pl.when(act)
    def _p1():
        pltpu.sync_copy(S_SRCA, srcA_v)
        pltpu.sync_copy(S_SRCB, srcB_v)
        agg_pass(wws, wwd, www, R_WW // NSUB, R_WW // NSUB // 8,
                 [srcA_v, srcB_v], [S_ACCA, S_ACCB], S_CNTWW)

    plsc.subcore_barrier()

    @pl.when(act & (wid < 10))
    def _p2():
        cvv = cv_v[ds(0, 16)]
        c2 = cvv[0]
        cFc = cvv[1]
        base = wid * 4000
        pltpu.sync_copy(S_CNTWW.at[ds(base, 4000)], fb2)
        pltpu.sync_copy(S_ACCA.at[ds(base, 4000)], fb0)

        def d1(i, _):
            sl = ds(i * 16, 16)
            fb1[sl] = fb0[sl] / jnp.maximum(fb2[sl], 1.0) + c2
            return 0
        lax.fori_loop(0, 250, d1, 0)
        pltpu.sync_copy(fb1, S_SRC2.at[ds(base, 4000)])
        pltpu.sync_copy(S_ACCB.at[ds(base, 4000)], fb0)

        def d2(i, _):
            sl = ds(i * 16, 16)
            fb1[sl] = fb0[sl] / jnp.maximum(fb2[sl], 1.0) + cFc
            return 0
        lax.fori_loop(0, 250, d2, 0)
        pltpu.sync_copy(fb1, S_SRCWT.at[ds(base, 4000)])

    plsc.subcore_barrier()

    @pl.when(act)
    def _p3():
        pltpu.sync_copy(S_SRC2, srcA_v)
        agg_pass(wws, wwd, www, R_WW // NSUB, R_WW // NSUB // 8,
                 [srcA_v], [S_ACC2], None)
        pltpu.sync_copy(S_SRCWT, srcA_v)
        agg_pass(wts, wtd, wtw, R_WT // NSUB, R_WT // NSUB // 8,
                 [srcA_v], [S_ACCWT], S_CNTWT)
        pltpu.sync_copy(S_SRCTT, srcT_v.at[ds(0, 800)])
        agg_pass(tts, ttd, ttw, R_TT // NSUB, R_TT // NSUB // 8,
                 [srcT_v], [S_ACCTT], S_CNTTT)

    plsc.subcore_barrier()

    @pl.when(act & (wid < 10))
    def _p4w():
        cAc = cv_v[ds(0, 16)][3]
        base = wid * 4000
        pltpu.sync_copy(S_CNTWW.at[ds(base, 4000)], fb2)
        pltpu.sync_copy(S_ACC2.at[ds(base, 4000)], fb0)

        def d1(i, _):
            sl = ds(i * 16, 16)
            fb1[sl] = fb0[sl] / jnp.maximum(fb2[sl], 1.0) + cAc
            return 0
        lax.fori_loop(0, 250, d1, 0)
        pltpu.sync_copy(fb1, S_SRCW.at[ds(base, 4000)])

    @pl.when(act & (wid == 10))
    def _p4t():
        cEc = cv_v[ds(0, 16)][2]
        pltpu.sync_copy(S_ACCWT.at[ds(0, 800)], fb0.at[ds(0, 800)])
        pltpu.sync_copy(S_CNTWT.at[ds(0, 800)], fb2.at[ds(0, 800)])

        def d1(i, _):
            sl = ds(i * 16, 16)
            fb1[sl] = fb0[sl] / jnp.maximum(fb2[sl], 1.0)
            return 0
        lax.fori_loop(0, 50, d1, 0)
        pltpu.sync_copy(S_ACCTT.at[ds(0, 800)], fb0.at[ds(0, 800)])
        pltpu.sync_copy(S_CNTTT.at[ds(0, 800)], fb2.at[ds(0, 800)])

        def d2(i, _):
            sl = ds(i * 16, 16)
            fb1[sl] = fb1[sl] + fb0[sl] / jnp.maximum(fb2[sl], 1.0) + cEc
            return 0
        lax.fori_loop(0, 50, d2, 0)
        pltpu.sync_copy(fb1.at[ds(0, 800)], S_SRCT)

    plsc.subcore_barrier()

    @pl.when(act)
    def _p5():
        pltpu.sync_copy(S_SRCW, srcA_v)
        agg_pass(wds, wdd, wdw, R_WD // NSUB, R_WD // NSUB // 8,
                 [srcA_v], [S_ACCD], S_CNTWD)
        pltpu.sync_copy(S_SRCT, srcT_v.at[ds(0, 800)])
        agg_pass(tds, tdd, tdw, R_TD // NSUB, R_TD // NSUB // 8,
                 [srcT_v], [S_ACCTD], S_CNTTD)

    plsc.subcore_barrier()

    @pl.when(act & (wid < 8))
    def _p6():
        base = wid * 400
        pltpu.sync_copy(S_ACCD.at[ds(base, 400)], fb0.at[ds(0, 400)])
        pltpu.sync_copy(S_CNTWD.at[ds(base, 400)], fb2.at[ds(0, 400)])

        def d1(i, _):
            sl = ds(i * 16, 16)
            fb1[sl] = fb0[sl] / jnp.maximum(fb2[sl], 1.0)
            return 0
        lax.fori_loop(0, 25, d1, 0)
        pltpu.sync_copy(S_ACCTD.at[ds(base, 400)], fb0.at[ds(0, 400)])
        pltpu.sync_copy(S_CNTTD.at[ds(base, 400)], fb2.at[ds(0, 400)])

        def d2(i, _):
            sl = ds(i * 16, 16)
            fb1[sl] = fb1[sl] + fb0[sl] / jnp.maximum(fb2[sl], 1.0)
            return 0
        lax.fori_loop(0, 25, d2, 0)
        pltpu.sync_copy(fb1.at[ds(0, 400)], out_h.at[ds(base, 400)])


_sc_call = functools.partial(
    pl.kernel,
    mesh=plsc.VectorSubcoreMesh(core_axis_name="c", subcore_axis_name="s"),
    out_type=jax.ShapeDtypeStruct((Nd,), jnp.float32),
    compiler_params=pltpu.CompilerParams(needs_layout_passes=False),
    scratch_types=[
        pltpu.VMEM((Nw,), jnp.float32),
        pltpu.VMEM((Nw,), jnp.float32),
        pltpu.VMEM((800,), jnp.float32),
        pltpu.VMEM((128,), jnp.float32),
        pltpu.VMEM((AD,), jnp.float32),
        pltpu.VMEM((4000,), jnp.int32),
        pltpu.VMEM((4000,), jnp.float32),
        pltpu.VMEM((4000,), jnp.float32),
        pltpu.VMEM((4000,), jnp.float32),
        pltpu.VMEM((8, 128), jnp.int32),
        pltpu.VMEM((8, 128), jnp.int32),
        pltpu.VMEM((8, 128), jnp.float32),
        pltpu.VMEM((8, 128), jnp.float32),
        pltpu.VMEM((8, 128), jnp.float32),
        pltpu.VMEM((128,), jnp.float32),
        pltpu.VMEM_SHARED((Nw,), jnp.float32),
        pltpu.VMEM_SHARED((Nw,), jnp.float32),
        pltpu.VMEM_SHARED((Nw,), jnp.float32),
        pltpu.VMEM_SHARED((Nw,), jnp.float32),
        pltpu.VMEM_SHARED((Nw,), jnp.float32),
        pltpu.VMEM_SHARED((AW,), jnp.float32),
        pltpu.VMEM_SHARED((AW,), jnp.float32),
        pltpu.VMEM_SHARED((AW,), jnp.float32),
        pltpu.VMEM_SHARED((AW,), jnp.float32),
        pltpu.VMEM_SHARED((800,), jnp.float32),
        pltpu.VMEM_SHARED((AT,), jnp.float32),
        pltpu.VMEM_SHARED((AT,), jnp.float32),
        pltpu.VMEM_SHARED((AT,), jnp.float32),
        pltpu.VMEM_SHARED((AT,), jnp.float32),
        pltpu.VMEM_SHARED((800,), jnp.float32),
        pltpu.VMEM_SHARED((AD,), jnp.float32),
        pltpu.VMEM_SHARED((AD,), jnp.float32),
        pltpu.VMEM_SHARED((AD,), jnp.float32),
        pltpu.VMEM_SHARED((AD,), jnp.float32),
    ],
)(_sc_body)


def _pad_edges(src, dst, w, rows, dummy):
    E = src.shape[0]
    P = rows * 128 - E
    s = jnp.concatenate([src.astype(jnp.int32),
                         jnp.zeros((P,), jnp.int32)]).reshape(rows, 128)
    d = jnp.concatenate([dst.astype(jnp.int32),
                         jnp.full((P,), dummy, jnp.int32)]).reshape(rows, 128)
    ww = jnp.concatenate([w, jnp.zeros((P,), jnp.float32)]).reshape(rows, 128)
    return s, d, ww


def kernel(word_id, topic_id, src_ww, dst_ww, w_ww, src_wt, dst_wt, w_wt,
           src_wd, dst_wd, w_wd, src_td, dst_td, w_td, src_tt, dst_tt, w_tt,
           word_embeds, topic_embeds, l1_Www, l1_bww, l1_Wwt, l1_bwt,
           l1_Wwd, l1_bwd, l1_Wtd, l1_btd, l1_Wtt, l1_btt, l2_Www, l2_bww,
           l2_Wwt, l2_bwt, l2_Wwd, l2_bwd, l2_Wtd, l2_btd, l2_Wtt, l2_btt,
           out_W, out_b, y_data):
    f32 = jnp.float32
    te = jnp.pad(topic_embeds, ((0, NTOPIC - topic_embeds.shape[0]), (0, 0)))
    r = lambda b: b.reshape(1, 128)

    wpa, wpb, tpp, cvec = pl.pallas_call(
        _pre_body,
        out_shape=[
            jax.ShapeDtypeStruct((NWORD, 1), f32),
            jax.ShapeDtypeStruct((NWORD, 1), f32),
            jax.ShapeDtypeStruct((NTOPIC, 1), f32),
            jax.ShapeDtypeStruct((1, 128), f32),
        ],
    )(word_embeds, te, l1_Www, r(l1_bww), l1_Wwt, r(l1_bwt), l1_Wwd,
      r(l1_bwd), l1_Wtd, r(l1_btd), l1_Wtt, r(l1_btt), l2_Www, r(l2_bww),
      l2_Wwt, r(l2_bwt), l2_Wwd, r(l2_bwd), l2_Wtd, r(l2_btd), l2_Wtt,
      r(l2_btt), out_W)

    wws, wwd2, www2 = _pad_edges(src_ww, dst_ww, w_ww, R_WW, DUM_W)
    wts, wtd2, wtw2 = _pad_edges(src_wt, dst_wt, w_wt, R_WT, DUM_T)
    wds, wdd2, wdw2 = _pad_edges(src_wd, dst_wd, w_wd, R_WD, DUM_D)
    tds, tdd2, tdw2 = _pad_edges(src_td, dst_td, w_td, R_TD, DUM_D)
    tts, ttd2, ttw2 = _pad_edges(src_tt, dst_tt, w_tt, R_TT, DUM_T)

    sd = _sc_call(
        wpa.reshape(NWORD), wpb.reshape(NWORD), tpp.reshape(NTOPIC),
        cvec.reshape(128), word_id.astype(jnp.int32),
        topic_id.astype(jnp.int32),
        wws, wwd2, www2, wts, wtd2, wtw2, wds, wdd2, wdw2,
        tds, tdd2, tdw2, tts, ttd2, ttw2)

    loss2, probs2 = pl.pallas_call(
        _post_body,
        out_shape=[
            jax.ShapeDtypeStruct((1, 1), f32),
            jax.ShapeDtypeStruct((B, 1), f32),
        ],
    )(sd.reshape(B, Nd // B), y_data.reshape(B, 1), out_b.reshape(1, 1))

    return loss2[0, 0], probs2.reshape(B)
